# baseline (speedup 1.0000x reference)
import functools
import math

import jax
import jax.numpy as jnp
from jax import lax
from jax.experimental import pallas as pl
from jax.experimental.pallas import tpu as pltpu
from jax.experimental.pallas import tpu_sc as plsc

DIM = 64
SCALE = math.sqrt(DIM)
CHUNK = 128
NSLOT = 4
AHEAD = 2


@functools.cache
def _make_sc_lookup(n_rows: int):
    info = plsc.get_sparse_core_info()
    nw = info.num_cores * info.num_subcores
    rows_per_w = n_rows // nw
    assert rows_per_w * nw == n_rows
    nch = rows_per_w // CHUNK
    assert nch * CHUNK == rows_per_w and nch >= 2 * NSLOT and nch % NSLOT == 0

    mesh = plsc.VectorSubcoreMesh(core_axis_name="c", subcore_axis_name="s")

    @functools.partial(
        pl.kernel,
        out_type=jax.ShapeDtypeStruct((n_rows, DIM), jnp.float32),
        mesh=mesh,
        compiler_params=pltpu.CompilerParams(use_tc_tiling_on_sc=False),
        scratch_types=[
            pltpu.VMEM((nch, CHUNK), jnp.int32),
            pltpu.VMEM((NSLOT, CHUNK, DIM), jnp.float32),
            pltpu.SemaphoreType.DMA,
            pltpu.SemaphoreType.DMA,
            pltpu.SemaphoreType.DMA,
            pltpu.SemaphoreType.DMA,
            pltpu.SemaphoreType.DMA,
            pltpu.SemaphoreType.DMA,
            pltpu.SemaphoreType.DMA,
            pltpu.SemaphoreType.DMA,
        ],
    )
    def lookup(idx_hbm, table_hbm, out_hbm, idx_v, bufs,
               sg0, sg1, sg2, sg3, so0, so1, so2, so3):
        gsem = (sg0, sg1, sg2, sg3)
        osem = (so0, so1, so2, so3)

        wid = lax.axis_index("s") * info.num_cores + lax.axis_index("c")
        base_row = wid * rows_per_w

        pltpu.sync_copy(idx_hbm.at[pl.ds(wid * nch, nch)], idx_v)

        def start_gather(gb, b):
            pltpu.async_copy(table_hbm.at[idx_v.at[gb]], bufs.at[b],
                             gsem[b])

        def wait_gather(b):
            pltpu.make_async_copy(
                table_hbm.at[pl.ds(0, CHUNK)], bufs.at[b], gsem[b]).wait()

        def start_out(gb, b):
            row0 = base_row + gb * CHUNK
            pltpu.async_copy(bufs.at[b], out_hbm.at[pl.ds(row0, CHUNK)],
                             osem[b])

        def wait_out(b):
            pltpu.make_async_copy(
                bufs.at[b], out_hbm.at[pl.ds(0, CHUNK)], osem[b]).wait()

        for gb in range(AHEAD):
            start_gather(gb, gb)

        @pl.loop(0, nch)
        def _(g):
            for b in range(NSLOT):
                @pl.when(g % NSLOT == b)
                def _():
                    wait_gather(b)
                    start_out(g, b)
                    b2 = (b + AHEAD) % NSLOT
                    @pl.when(g >= AHEAD)
                    def _():
                        wait_out(b2)

                    @pl.when(g + AHEAD < nch)
                    def _():
                        start_gather(g + AHEAD, b2)

        for gb in range(nch - AHEAD, nch):
            wait_out(gb % NSLOT)

    return lookup


def kernel(x, table):
    rows, cols = x.shape
    n = rows * cols
    idx = x.reshape(n // CHUNK, CHUNK).astype(jnp.int32)
    out = _make_sc_lookup(n)(idx, table)
    return out.reshape(rows, cols, DIM) * SCALE
